# baseline (speedup 1.0000x reference)
import jax
import jax.numpy as jnp
from jax.experimental import pallas as pl
from jax.experimental.pallas import tpu as pltpu

_L = 16
_ROWS = 512
_COLS = 128


def _gate_body(idx_smem, vals_hbm, out_ref, vals_v, sem):
    copy = pltpu.make_async_copy(vals_hbm, vals_v, sem)
    copy.start()
    lane = jax.lax.broadcasted_iota(jnp.int32, (1, _COLS), 1)
    rows = [idx_smem[i] // _COLS for i in range(_L)]
    shifts = [(i - idx_smem[i]) % _COLS for i in range(_L)]
    copy.wait()
    parts = []
    for i in range(_L):
        vrow = vals_v[pl.ds(rows[i], 1), :]
        rolled = pltpu.roll(vrow, shifts[i], 1)
        parts.append(jnp.where(lane == i, rolled, 1.0))
    while len(parts) > 1:
        parts = [a * b for a, b in zip(parts[::2], parts[1::2])]
    lg = jnp.log(jnp.maximum(parts[0], 1e-38))
    total = jax.lax.dot_general(
        lg,
        jnp.ones((_COLS, 1), jnp.float32),
        (((1,), (0,)), ((), ())),
        precision=jax.lax.Precision.HIGHEST,
        preferred_element_type=jnp.float32,
    )
    out_ref[0] = jnp.exp(total)[0, 0]


@jax.jit
def _gate(vals, idx):
    return pl.pallas_call(
        _gate_body,
        in_specs=[
            pl.BlockSpec(memory_space=pltpu.SMEM),
            pl.BlockSpec(memory_space=pltpu.MemorySpace.HBM),
        ],
        out_specs=pl.BlockSpec(memory_space=pltpu.SMEM),
        out_shape=jax.ShapeDtypeStruct((1,), jnp.float32),
        scratch_shapes=[
            pltpu.VMEM((_ROWS, _COLS), jnp.float32),
            pltpu.SemaphoreType.DMA,
        ],
    )(idx, vals.reshape(_ROWS, _COLS))


def kernel(input_values, input_idxs):
    out = _gate(input_values, input_idxs.astype(jnp.int32))
    return out.reshape(())
